# baseline (speedup 1.0000x reference)
import functools

import jax
import jax.numpy as jnp
from jax import lax
from jax.experimental import pallas as pl
from jax.experimental.pallas import tpu as pltpu
from jax.experimental.pallas import tpu_sc as plsc

BATCH = 16384
DIM = 64
NC = 2
NS = 16
NW = NC * NS
BPW = BATCH // NW
LANES = 16
NGROUPS = BPW // LANES

NE = 1000000
WIN = 1024
NWIN = 977
LAST_WIN = NWIN - 1
LAST_WIN_SIZE = NE - LAST_WIN * WIN
NSLOT = 2 * BATCH
ROWS_PAD = NSLOT + 8
PK_PAD = 31744 * 65536
STRIP = 4096

_mesh = plsc.VectorSubcoreMesh(core_axis_name="c", subcore_axis_name="s")


def _sqrt16(x):
    i = plsc.bitcast(x, jnp.int32)
    i = jnp.int32(0x5F3759DF) - lax.shift_right_arithmetic(i, 1)
    y = plsc.bitcast(i, jnp.float32)
    for _ in range(3):
        y = y * (jnp.float32(1.5) - jnp.float32(0.5) * x * y * y)
    return x * y


@functools.partial(
    pl.kernel,
    out_type=jax.ShapeDtypeStruct((NSLOT,), jnp.int32),
    mesh=_mesh,
    compiler_params=pltpu.CompilerParams(
        needs_layout_passes=False, use_tc_tiling_on_sc=False),
    scratch_types=[
        pltpu.VMEM((BPW,), jnp.int32),
        pltpu.VMEM((BPW,), jnp.int32),
        pltpu.VMEM((BPW,), jnp.int32),
        pltpu.VMEM((BPW,), jnp.int32),
        pltpu.SemaphoreType.DMA,
        pltpu.SemaphoreType.DMA,
    ],
)
def _ids_kernel(xs_hbm, ys_hbm, gid_hbm, ids_hbm, xv, yv, xe, ye, sem1, sem2):
    wid = lax.axis_index("s") * NC + lax.axis_index("c")
    base = wid * BPW
    pltpu.sync_copy(xs_hbm.at[pl.ds(base, BPW)], xv)
    pltpu.sync_copy(ys_hbm.at[pl.ds(base, BPW)], yv)
    cx = pltpu.async_copy(gid_hbm.at[xv], xe, sem1)
    cy = pltpu.async_copy(gid_hbm.at[yv], ye, sem2)
    cx.wait()
    cy.wait()
    pltpu.sync_copy(xe, ids_hbm.at[pl.ds(base, BPW)])
    pltpu.sync_copy(ye, ids_hbm.at[pl.ds(BATCH + base, BPW)])


@functools.partial(
    pl.kernel,
    out_type=jax.ShapeDtypeStruct((ROWS_PAD, 128), jnp.float32),
    mesh=_mesh,
    compiler_params=pltpu.CompilerParams(
        needs_layout_passes=False, use_tc_tiling_on_sc=True),
    scratch_types=[
        pltpu.VMEM((DIM, WIN), jnp.float32),
        pltpu.VMEM((NSLOT + LANES,), jnp.int32),
        pltpu.VMEM((8192 + LANES,), jnp.int32),
        pltpu.VMEM((STRIP,), jnp.int32),
        pltpu.VMEM((LANES, 128), jnp.float32),
        pltpu.SemaphoreType.DMA,
    ],
)
def _rows_kernel(ids_hbm, embt_hbm, tail_hbm, rows_hbm, chunk, sel, clist,
                 strip, stag, sem):
    wid = lax.axis_index("s") * NC + lax.axis_index("c")
    wstart = wid * 30 + jnp.minimum(wid, 17)
    wcount = 30 + jnp.where(wid < 17, 1, 0)
    e0 = wstart * WIN
    lane_iota = lax.iota(jnp.int32, LANES)
    zeros16 = jnp.zeros((LANES,), jnp.float32)

    for d in range(DIM, 128):
        plsc.store_scatter(stag, [lane_iota, jnp.full((LANES,), d, jnp.int32)],
                           zeros16)

    def strip_body(s, cnt):
        pltpu.sync_copy(ids_hbm.at[pl.ds(s * STRIP, STRIP)], strip)

        def vreg_body(k, cnt):
            e = strip[pl.ds(k * LANES, LANES)]
            le = e - e0
            m = (le >= 0) & (le < wcount * WIN)
            slot = s * STRIP + k * LANES + lane_iota
            pk = lax.shift_left(le, 16) + slot
            plsc.store_compressed(sel.at[pl.ds(cnt, LANES)], pk, mask=m)
            return cnt + plsc.all_reduce_population_count(m)[0]

        return lax.fori_loop(0, STRIP // LANES, vreg_body, cnt)

    nsel = lax.fori_loop(0, NSLOT // STRIP, strip_body, jnp.int32(0))
    sel[pl.ds(nsel, LANES)] = jnp.full((LANES,), PK_PAD, jnp.int32)
    nselv = pl.cdiv(nsel, LANES)

    def win_body(c, carry):
        gw = wstart + c

        @pl.when(gw != LAST_WIN)
        def _():
            pltpu.sync_copy(embt_hbm.at[:, pl.ds(gw * WIN, WIN)], chunk)

        @pl.when(gw == LAST_WIN)
        def _():
            pltpu.sync_copy(embt_hbm.at[:, pl.ds(LAST_WIN * WIN, 512)],
                            chunk.at[:, pl.ds(0, 512)])
            pltpu.sync_copy(tail_hbm, chunk.at[:, pl.ds(512, 128)])

        def match_body(k, mcnt):
            pk = sel[pl.ds(k * LANES, LANES)]
            m = lax.shift_right_arithmetic(pk, 26) == c
            plsc.store_compressed(clist.at[pl.ds(mcnt, LANES)], pk, mask=m)
            return mcnt + plsc.all_reduce_population_count(m)[0]

        mcnt = lax.fori_loop(0, nselv, match_body, jnp.int32(0))
        clist[pl.ds(mcnt, LANES)] = jnp.full((LANES,), NSLOT, jnp.int32)

        def extract_body(k, carry2):
            pk = clist[pl.ds(k * LANES, LANES)]
            slot = pk & jnp.int32(0xFFFF)
            col = lax.shift_right_arithmetic(pk, 16) & jnp.int32(WIN - 1)
            for d in range(DIM):
                vals = plsc.load_gather(
                    chunk, [jnp.full((LANES,), d, jnp.int32), col])
                plsc.store_scatter(
                    stag, [lane_iota, jnp.full((LANES,), d, jnp.int32)], vals)
            pltpu.async_copy(stag, rows_hbm.at[slot], sem).wait()
            return carry2

        lax.fori_loop(0, pl.cdiv(mcnt, LANES), extract_body, 0)
        return carry

    lax.fori_loop(0, wcount, win_body, 0)


@functools.partial(
    pl.kernel,
    out_type=jax.ShapeDtypeStruct((BATCH,), jnp.float32),
    mesh=_mesh,
    compiler_params=pltpu.CompilerParams(
        needs_layout_passes=False, use_tc_tiling_on_sc=True),
    scratch_types=[
        pltpu.VMEM((128, 128), jnp.float32),
        pltpu.VMEM((128, 128), jnp.float32),
        pltpu.VMEM((DIM,), jnp.float32),
        pltpu.VMEM((BPW,), jnp.float32),
        pltpu.SemaphoreType.DMA,
        pltpu.SemaphoreType.DMA,
    ],
)
def _score_kernel(rows_hbm, rel_hbm, out_hbm, hv, tv, rv, ov, sem1, sem2):
    wid = lax.axis_index("s") * NC + lax.axis_index("c")
    base = wid * BPW
    pltpu.sync_copy(rel_hbm, rv)
    rchunks = [rv[pl.ds(j * LANES, LANES)] for j in range(DIM // LANES)]
    lane_iota = lax.iota(jnp.int32, LANES)
    perms = {d: lane_iota ^ d for d in (1, 2, 4, 8)}
    masks = {d: (lane_iota & d) == 0 for d in (1, 2, 4, 8)}

    def combine(a, b, dist):
        m = masks[dist]
        w = jnp.where(m, b, a)
        _, wp = plsc.sort_key_val(perms[dist], w)
        return jnp.where(m, a, b) + wp

    def sub_body(j, carry):
        s0 = base + j * 128
        ch = pltpu.async_copy(rows_hbm.at[pl.ds(s0, 128), :], hv, sem1)
        ct = pltpu.async_copy(rows_hbm.at[pl.ds(BATCH + s0, 128), :], tv, sem2)
        ch.wait()
        ct.wait()

        def group_body(g, carry2):
            svecs = []
            for p in range(LANES):
                i = g * LANES + p
                s = None
                for q in range(DIM // LANES):
                    hq = hv[i, pl.ds(q * LANES, LANES)]
                    tq = tv[i, pl.ds(q * LANES, LANES)]
                    dd = hq - tq + rchunks[q]
                    s = dd * dd if s is None else s + dd * dd
                svecs.append(s)
            dist = 1
            while len(svecs) > 1:
                svecs = [combine(svecs[k], svecs[k + 1], dist)
                         for k in range(0, len(svecs), 2)]
                dist *= 2
            ov[pl.ds(j * 128 + g * LANES, LANES)] = _sqrt16(svecs[0])
            return carry2

        lax.fori_loop(0, 128 // LANES, group_body, 0)
        return carry

    lax.fori_loop(0, BPW // 128, sub_body, 0)
    pltpu.sync_copy(ov, out_hbm.at[pl.ds(base, BPW)])


def kernel(data, graph_ids, entity_emb, relation_emb):
    xs = data[:, 0]
    ys = data[:, 1]
    embt = entity_emb.T
    tail = jnp.pad(entity_emb[LAST_WIN * WIN + 512:].T, ((0, 0), (0, 64)))
    rel = relation_emb.reshape(DIM)
    ids = _ids_kernel(xs, ys, graph_ids)
    rows = _rows_kernel(ids, embt, tail)
    scores = _score_kernel(rows, rel)
    return scores.reshape(BATCH, 1)
